# Initial kernel scaffold; baseline (speedup 1.0000x reference)
import jax
import jax.numpy as jnp
from jax.experimental import pallas as pl


def kernel(x, edge_index, params):
    raise NotImplementedError("write your pallas kernel here")


# speedup vs baseline: 11.7803x; 11.7803x over previous
import functools

import jax
import jax.numpy as jnp
from jax import lax
from jax.experimental import pallas as pl
from jax.experimental.pallas import tpu as pltpu
from jax.experimental.pallas import tpu_sc as plsc

N_NODES = 10000
N_EDGES = 320000
D = 128
G = 64

NC, NS, L = 2, 16, 16
NW = NC * NS
NPAD = 10240
NPT = NPAD // NW
EPT = N_EDGES // NW
GC = 80

_mesh = plsc.VectorSubcoreMesh(core_axis_name="c", subcore_axis_name="s")


def _wid():
    return lax.axis_index("s") * NC + lax.axis_index("c")


@functools.partial(
    pl.kernel,
    out_type=jax.ShapeDtypeStruct((NW * NPAD,), jnp.int32),
    mesh=_mesh,
    compiler_params=pltpu.CompilerParams(needs_layout_passes=False),
    scratch_types=[
        pltpu.VMEM((EPT,), jnp.int32),
        pltpu.VMEM((NPAD,), jnp.int32),
        pltpu.VMEM((L,), jnp.int32),
    ],
)
def _win_tables(dst_hbm, tables_hbm, dst_v, win_v, tmp_v):
    wid = _wid()
    base_e = wid * EPT
    pltpu.sync_copy(dst_hbm.at[pl.ds(base_e, EPT)], dst_v)

    lane = lax.iota(jnp.int32, L)
    neg1 = jnp.full((L,), -1, jnp.int32)

    def init_body(i, _):
        win_v[pl.ds(i * L, L)] = neg1
        return 0

    lax.fori_loop(0, NPAD // L, init_body, 0)

    def body(c, _):
        d16 = dst_v[pl.ds(c * L, L)]
        ckey = d16 * L + lane
        e16 = base_e + c * L + lane
        sk, se = plsc.sort_key_val(ckey, e16)
        sd = lax.shift_right_logical(sk, 4)
        tmp_v[...] = sd
        nxt = plsc.load_gather(tmp_v, [jnp.minimum(lane + 1, L - 1)])
        winner = (sd != nxt) | (lane == L - 1)
        cur = plsc.load_gather(win_v, [sd])
        plsc.store_scatter(win_v, [sd], jnp.maximum(cur, se), mask=winner)
        return 0

    lax.fori_loop(0, EPT // L, body, 0)
    pltpu.sync_copy(win_v, tables_hbm.at[pl.ds(wid * NPAD, NPAD)])


@functools.partial(
    pl.kernel,
    out_type=jax.ShapeDtypeStruct((NPAD, D), jnp.float32),
    mesh=_mesh,
    scratch_types=[
        pltpu.VMEM((NW * NPT,), jnp.int32),
        pltpu.VMEM((GC,), jnp.int32),
        pltpu.VMEM((NPT, D), jnp.float32),
        pltpu.SemaphoreType.DMA,
    ],
)
def _merge_gather(tables_hbm, src_hbm, x_hbm, xs_hbm, tabs_v, sidx_v,
                  xrows_v, sem):
    wid = _wid()
    base_n = wid * NPT
    for t in range(NW):
        pltpu.sync_copy(tables_hbm.at[pl.ds(t * NPAD + base_n, NPT)],
                        tabs_v.at[pl.ds(t * NPT, NPT)])

    def merge_body(i, _):
        m = tabs_v[pl.ds(i * L, L)]
        for t in range(1, NW):
            m = jnp.maximum(m, tabs_v[pl.ds(t * NPT + i * L, L)])
        tabs_v[pl.ds(i * L, L)] = jnp.maximum(m, 0)
        return 0

    lax.fori_loop(0, NPT // L, merge_body, 0)

    def gather_body(c, _):
        pltpu.async_copy(src_hbm.at[tabs_v.at[pl.ds(c * GC, GC)]],
                         sidx_v, sem).wait()
        pltpu.async_copy(x_hbm.at[sidx_v],
                         xrows_v.at[pl.ds(c * GC, GC)], sem).wait()
        return 0

    lax.fori_loop(0, NPT // GC, gather_body, 0)
    pltpu.sync_copy(xrows_v, xs_hbm.at[pl.ds(base_n, NPT)])


def _dense_body(x_ref, xs_ref, wq, bq, wk, bk, wv, bv, wr, br, gma, bta, ws,
                bs, wsc, out_ref):
    xs = xs_ref[...]
    xx = x_ref[...]
    q = jnp.dot(xs, wq[...], preferred_element_type=jnp.float32) + bq[...]
    k = jnp.dot(xx, wk[...], preferred_element_type=jnp.float32) + bk[...]
    v = jnp.dot(xx, wv[...], preferred_element_type=jnp.float32) + bv[...]
    scores = (q * k) * (1.0 / 8.0)
    scores = scores - jnp.max(scores, axis=-1, keepdims=True)
    ex = jnp.exp(scores)
    attn = ex / jnp.sum(ex, axis=-1, keepdims=True)
    attn_out = attn * v
    mu = jnp.mean(attn_out, axis=-1, keepdims=True)
    ctr = attn_out - mu
    var = jnp.mean(ctr * ctr, axis=-1, keepdims=True)
    xn = ctr * lax.rsqrt(var + 1e-5) * gma[...] + bta[...]
    h = attn_out + jnp.maximum(
        jnp.dot(xn, wr[...], preferred_element_type=jnp.float32) + br[...], 0.0)
    gate = jnp.dot(h, ws[...], preferred_element_type=jnp.float32) + bs[...] \
        + attn_out
    out_ref[...] = xs - xx + jnp.dot(gate, wsc[...],
                                     preferred_element_type=jnp.float32)


def _dense(x_pad, xs, hp, w_scale):
    blk = 1280
    grid = NPAD // blk

    def row_spec(dim):
        return pl.BlockSpec((blk, dim), lambda i: (i, 0))

    def full_spec(a):
        return pl.BlockSpec(a.shape, lambda i: (0,) * a.ndim)

    weights = [hp['Wq'], hp['bq'].reshape(1, G), hp['Wk'], hp['bk'].reshape(1, G),
               hp['Wv'], hp['bv'].reshape(1, G), hp['res'][0]['W'],
               hp['res'][0]['b'].reshape(1, G), hp['res'][0]['gamma'].reshape(1, G),
               hp['res'][0]['beta'].reshape(1, G), hp['Ws'],
               hp['bs'].reshape(1, G), w_scale]
    return pl.pallas_call(
        _dense_body,
        out_shape=jax.ShapeDtypeStruct((NPAD, D), jnp.float32),
        grid=(grid,),
        in_specs=[row_spec(D), row_spec(D)] + [full_spec(w) for w in weights],
        out_specs=row_spec(D),
    )(x_pad, xs, *weights)


@functools.partial(
    pl.kernel,
    out_type=jax.ShapeDtypeStruct((N_EDGES, D), jnp.float32),
    mesh=_mesh,
    scratch_types=[
        pltpu.VMEM((EPT,), jnp.int32),
        pltpu.VMEM((GC, D), jnp.float32),
        pltpu.SemaphoreType.DMA,
    ],
)
def _edge_gather(h_hbm, dst_hbm, out_hbm, dst_v, rows_v, sem):
    wid = _wid()
    base_e = wid * EPT
    pltpu.sync_copy(dst_hbm.at[pl.ds(base_e, EPT)], dst_v)

    def body(c, _):
        pltpu.async_copy(h_hbm.at[dst_v.at[pl.ds(c * GC, GC)]],
                         rows_v, sem).wait()
        pltpu.sync_copy(rows_v, out_hbm.at[pl.ds(base_e + c * GC, GC)])
        return 0

    lax.fori_loop(0, EPT // GC, body, 0)


def kernel(x, edge_index, params):
    src = edge_index[0]
    dst = edge_index[1]
    hp = params['heads'][0]
    x_pad = jnp.concatenate(
        [x, jnp.zeros((NPAD - N_NODES, D), jnp.float32)], axis=0)

    tables = _win_tables(dst)
    xs = _merge_gather(tables, src, x)
    h_node = _dense(x_pad, xs, hp, params['W_scale'])
    return _edge_gather(h_node, dst)


# speedup vs baseline: 20.8088x; 1.7664x over previous
import functools

import jax
import jax.numpy as jnp
from jax import lax
from jax.experimental import pallas as pl
from jax.experimental.pallas import tpu as pltpu
from jax.experimental.pallas import tpu_sc as plsc

N_NODES = 10000
N_EDGES = 320000
D = 128
G = 64

NC, NS, L = 2, 16, 16
NW = NC * NS
NPAD = 10240
NPT = NPAD // NW
EPT = N_EDGES // NW
GC = 80

_mesh = plsc.VectorSubcoreMesh(core_axis_name="c", subcore_axis_name="s")


def _wid():
    return lax.axis_index("s") * NC + lax.axis_index("c")


@functools.partial(
    pl.kernel,
    out_type=jax.ShapeDtypeStruct((NW * NPAD,), jnp.int32),
    mesh=_mesh,
    compiler_params=pltpu.CompilerParams(needs_layout_passes=False),
    scratch_types=[
        pltpu.VMEM((EPT,), jnp.int32),
        pltpu.VMEM((NPAD,), jnp.int32),
        pltpu.VMEM((L,), jnp.int32),
    ],
)
def _win_tables(dst_hbm, tables_hbm, dst_v, win_v, tmp_v):
    wid = _wid()
    base_e = wid * EPT
    pltpu.sync_copy(dst_hbm.at[pl.ds(base_e, EPT)], dst_v)

    lane = lax.iota(jnp.int32, L)
    neg1 = jnp.full((L,), -1, jnp.int32)

    def init_body(i, _):
        win_v[pl.ds(i * L, L)] = neg1
        return 0

    lax.fori_loop(0, NPAD // L, init_body, 0)

    def body(c, _):
        d16 = dst_v[pl.ds(c * L, L)]
        ckey = d16 * L + lane
        e16 = base_e + c * L + lane
        sk, se = plsc.sort_key_val(ckey, e16)
        sd = lax.shift_right_logical(sk, 4)
        tmp_v[...] = sd
        nxt = plsc.load_gather(tmp_v, [jnp.minimum(lane + 1, L - 1)])
        winner = (sd != nxt) | (lane == L - 1)
        cur = plsc.load_gather(win_v, [sd])
        plsc.store_scatter(win_v, [sd], jnp.maximum(cur, se), mask=winner)
        return 0

    lax.fori_loop(0, EPT // L, body, 0)
    pltpu.sync_copy(win_v, tables_hbm.at[pl.ds(wid * NPAD, NPAD)])


@functools.partial(
    pl.kernel,
    out_type=jax.ShapeDtypeStruct((NPAD, D), jnp.float32),
    mesh=_mesh,
    scratch_types=[
        pltpu.VMEM((NW * NPT,), jnp.int32),
        pltpu.VMEM((GC,), jnp.int32),
        pltpu.VMEM((NPT, D), jnp.float32),
        pltpu.SemaphoreType.DMA,
    ],
)
def _merge_gather(tables_hbm, src_hbm, x_hbm, xs_hbm, tabs_v, sidx_v,
                  xrows_v, sem):
    wid = _wid()
    base_n = wid * NPT
    for t in range(NW):
        pltpu.sync_copy(tables_hbm.at[pl.ds(t * NPAD + base_n, NPT)],
                        tabs_v.at[pl.ds(t * NPT, NPT)])

    def merge_body(i, _):
        m = tabs_v[pl.ds(i * L, L)]
        for t in range(1, NW):
            m = jnp.maximum(m, tabs_v[pl.ds(t * NPT + i * L, L)])
        tabs_v[pl.ds(i * L, L)] = jnp.maximum(m, 0)
        return 0

    lax.fori_loop(0, NPT // L, merge_body, 0)

    def gather_body(c, _):
        pltpu.async_copy(src_hbm.at[tabs_v.at[pl.ds(c * GC, GC)]],
                         sidx_v, sem).wait()
        pltpu.async_copy(x_hbm.at[sidx_v],
                         xrows_v.at[pl.ds(c * GC, GC)], sem).wait()
        return 0

    lax.fori_loop(0, NPT // GC, gather_body, 0)
    pltpu.sync_copy(xrows_v, xs_hbm.at[pl.ds(base_n, NPT)])


def _dense_body(x_ref, xs_ref, wq, bq, wk, bk, wv, bv, wr, br, gma, bta, ws,
                bs, wsc, out_ref):
    xs = xs_ref[...]
    xx = x_ref[...]
    q = jnp.dot(xs, wq[...], preferred_element_type=jnp.float32) + bq[...]
    k = jnp.dot(xx, wk[...], preferred_element_type=jnp.float32) + bk[...]
    v = jnp.dot(xx, wv[...], preferred_element_type=jnp.float32) + bv[...]
    scores = (q * k) * (1.0 / 8.0)
    scores = scores - jnp.max(scores, axis=-1, keepdims=True)
    ex = jnp.exp(scores)
    attn = ex / jnp.sum(ex, axis=-1, keepdims=True)
    attn_out = attn * v
    mu = jnp.mean(attn_out, axis=-1, keepdims=True)
    ctr = attn_out - mu
    var = jnp.mean(ctr * ctr, axis=-1, keepdims=True)
    xn = ctr * lax.rsqrt(var + 1e-5) * gma[...] + bta[...]
    h = attn_out + jnp.maximum(
        jnp.dot(xn, wr[...], preferred_element_type=jnp.float32) + br[...], 0.0)
    gate = jnp.dot(h, ws[...], preferred_element_type=jnp.float32) + bs[...] \
        + attn_out
    out_ref[...] = xs - xx + jnp.dot(gate, wsc[...],
                                     preferred_element_type=jnp.float32)


def _dense(x_pad, xs, hp, w_scale):
    blk = 1280
    grid = NPAD // blk

    def row_spec(dim):
        return pl.BlockSpec((blk, dim), lambda i: (i, 0))

    def full_spec(a):
        return pl.BlockSpec(a.shape, lambda i: (0,) * a.ndim)

    weights = [hp['Wq'], hp['bq'].reshape(1, G), hp['Wk'], hp['bk'].reshape(1, G),
               hp['Wv'], hp['bv'].reshape(1, G), hp['res'][0]['W'],
               hp['res'][0]['b'].reshape(1, G), hp['res'][0]['gamma'].reshape(1, G),
               hp['res'][0]['beta'].reshape(1, G), hp['Ws'],
               hp['bs'].reshape(1, G), w_scale]
    return pl.pallas_call(
        _dense_body,
        out_shape=jax.ShapeDtypeStruct((NPAD, D), jnp.float32),
        grid=(grid,),
        in_specs=[row_spec(D), row_spec(D)] + [full_spec(w) for w in weights],
        out_specs=row_spec(D),
    )(x_pad, xs, *weights)


@functools.partial(
    pl.kernel,
    out_type=jax.ShapeDtypeStruct((N_EDGES, D), jnp.float32),
    mesh=_mesh,
    scratch_types=[
        pltpu.VMEM((EPT,), jnp.int32),
        pltpu.VMEM((GC, D), jnp.float32),
        pltpu.VMEM((GC, D), jnp.float32),
        pltpu.VMEM_SHARED((NPAD, D), jnp.float32),
        pltpu.SemaphoreType.DMA,
        pltpu.SemaphoreType.DMA,
        pltpu.SemaphoreType.DMA,
        pltpu.SemaphoreType.DMA,
        pltpu.SemaphoreType.DMA,
    ],
)
def _edge_gather(h_hbm, dst_hbm, out_hbm, dst_v, rows_a, rows_b, h_sh,
                 sem_l, sem_ga, sem_gb, sem_sa, sem_sb):
    wid = _wid()
    sid = lax.axis_index("s")
    base_e = wid * EPT
    rows_per_sub = NPAD // NS
    pltpu.async_copy(dst_hbm.at[pl.ds(base_e, EPT)], dst_v, sem_l)
    pltpu.sync_copy(h_hbm.at[pl.ds(sid * rows_per_sub, rows_per_sub)],
                    h_sh.at[pl.ds(sid * rows_per_sub, rows_per_sub)])
    pltpu.make_async_copy(dst_hbm.at[pl.ds(base_e, EPT)], dst_v, sem_l).wait()
    plsc.subcore_barrier()

    def g_start(c, buf, sem):
        pltpu.async_copy(h_sh.at[dst_v.at[pl.ds(c * GC, GC)]], buf, sem)

    def g_wait(buf, sem):
        pltpu.make_async_copy(h_sh.at[dst_v.at[pl.ds(0, GC)]], buf, sem).wait()

    def s_start(c, buf, sem):
        pltpu.async_copy(buf, out_hbm.at[pl.ds(base_e + c * GC, GC)], sem)

    def s_wait(buf, sem):
        pltpu.make_async_copy(buf, out_hbm.at[pl.ds(base_e, GC)], sem).wait()

    g_start(0, rows_a, sem_ga)

    def body(i, _):
        c = 2 * i

        @pl.when(i > 0)
        def _():
            s_wait(rows_b, sem_sb)

        g_start(c + 1, rows_b, sem_gb)
        g_wait(rows_a, sem_ga)
        s_start(c, rows_a, sem_sa)
        s_wait(rows_a, sem_sa)
        g_start(c + 2, rows_a, sem_ga)
        g_wait(rows_b, sem_gb)
        s_start(c + 1, rows_b, sem_sb)
        return 0

    n_pairs = (EPT // GC) // 2
    lax.fori_loop(0, n_pairs, body, 0)
    s_wait(rows_b, sem_sb)
    g_wait(rows_a, sem_ga)
    s_start(EPT // GC - 1, rows_a, sem_sa)
    s_wait(rows_a, sem_sa)


def kernel(x, edge_index, params):
    src = edge_index[0]
    dst = edge_index[1]
    hp = params['heads'][0]
    x_pad = jnp.concatenate(
        [x, jnp.zeros((NPAD - N_NODES, D), jnp.float32)], axis=0)

    tables = _win_tables(dst)
    xs = _merge_gather(tables, src, x)
    h_node = _dense(x_pad, xs, hp, params['W_scale'])
    return _edge_gather(h_node, dst)


# speedup vs baseline: 22.8924x; 1.1001x over previous
import functools

import jax
import jax.numpy as jnp
from jax import lax
from jax.experimental import pallas as pl
from jax.experimental.pallas import tpu as pltpu
from jax.experimental.pallas import tpu_sc as plsc

N_NODES = 10000
N_EDGES = 320000
D = 128
G = 64

NC, NS, L = 2, 16, 16
NW = NC * NS
NPAD = 10240
NPT = NPAD // NW
EPT = N_EDGES // NW
GC = 80

_mesh = plsc.VectorSubcoreMesh(core_axis_name="c", subcore_axis_name="s")


def _wid():
    return lax.axis_index("s") * NC + lax.axis_index("c")


@functools.partial(
    pl.kernel,
    out_type=jax.ShapeDtypeStruct((NW * NPAD,), jnp.int32),
    mesh=_mesh,
    compiler_params=pltpu.CompilerParams(needs_layout_passes=False),
    scratch_types=[
        pltpu.VMEM((EPT,), jnp.int32),
        pltpu.VMEM((NPAD,), jnp.int32),
        pltpu.VMEM((L,), jnp.int32),
    ],
)
def _win_tables(dst_hbm, tables_hbm, dst_v, win_v, tmp_v):
    wid = _wid()
    base_e = wid * EPT
    pltpu.sync_copy(dst_hbm.at[pl.ds(base_e, EPT)], dst_v)

    lane = lax.iota(jnp.int32, L)
    neg1 = jnp.full((L,), -1, jnp.int32)

    def init_body(i, _):
        win_v[pl.ds(i * L, L)] = neg1
        return 0

    lax.fori_loop(0, NPAD // L, init_body, 0)

    def body(c, _):
        d16 = dst_v[pl.ds(c * L, L)]
        ckey = d16 * L + lane
        e16 = base_e + c * L + lane
        sk, se = plsc.sort_key_val(ckey, e16)
        sd = lax.shift_right_logical(sk, 4)
        tmp_v[...] = sd
        nxt = plsc.load_gather(tmp_v, [jnp.minimum(lane + 1, L - 1)])
        winner = (sd != nxt) | (lane == L - 1)
        plsc.store_scatter(win_v, [sd], se, mask=winner)
        return 0

    lax.fori_loop(0, EPT // L, body, 0)
    pltpu.sync_copy(win_v, tables_hbm.at[pl.ds(wid * NPAD, NPAD)])


@functools.partial(
    pl.kernel,
    out_type=jax.ShapeDtypeStruct((NPAD, D), jnp.float32),
    mesh=_mesh,
    scratch_types=[
        pltpu.VMEM((NW * NPT,), jnp.int32),
        pltpu.VMEM((NPT,), jnp.int32),
        pltpu.VMEM((NPT, D), jnp.float32),
        pltpu.SemaphoreType.DMA,
        pltpu.SemaphoreType.DMA,
    ],
)
def _merge_gather(tables_hbm, src_hbm, x_hbm, xs_hbm, tabs_v, sidx_v,
                  xrows_v, sem_a, sem_b):
    wid = _wid()
    base_n = wid * NPT
    for t in range(NW):
        pltpu.async_copy(tables_hbm.at[pl.ds(t * NPAD + base_n, NPT)],
                         tabs_v.at[pl.ds(t * NPT, NPT)], sem_a)
    for t in range(NW):
        pltpu.make_async_copy(tables_hbm.at[pl.ds(base_n, NPT)],
                              tabs_v.at[pl.ds(0, NPT)], sem_a).wait()

    def merge_body(i, _):
        m = tabs_v[pl.ds(i * L, L)]
        for t in range(1, NW):
            m = jnp.maximum(m, tabs_v[pl.ds(t * NPT + i * L, L)])
        tabs_v[pl.ds(i * L, L)] = jnp.maximum(m, 0)
        return 0

    lax.fori_loop(0, NPT // L, merge_body, 0)

    for c in range(NPT // GC):
        pltpu.async_copy(src_hbm.at[tabs_v.at[pl.ds(c * GC, GC)]],
                         sidx_v.at[pl.ds(c * GC, GC)], sem_a)
    for c in range(NPT // GC):
        pltpu.make_async_copy(src_hbm.at[tabs_v.at[pl.ds(0, GC)]],
                              sidx_v.at[pl.ds(0, GC)], sem_a).wait()
    for c in range(NPT // GC):
        pltpu.async_copy(x_hbm.at[sidx_v.at[pl.ds(c * GC, GC)]],
                         xrows_v.at[pl.ds(c * GC, GC)], sem_b)
    for c in range(NPT // GC):
        pltpu.make_async_copy(x_hbm.at[sidx_v.at[pl.ds(0, GC)]],
                              xrows_v.at[pl.ds(0, GC)], sem_b).wait()
    pltpu.sync_copy(xrows_v, xs_hbm.at[pl.ds(base_n, NPT)])


def _dense_body(x_ref, xs_ref, wq, bq, wk, bk, wv, bv, wr, br, gma, bta, ws,
                bs, wsc, out_ref):
    xs = xs_ref[...]
    xx = x_ref[...]
    q = jnp.dot(xs, wq[...], preferred_element_type=jnp.float32) + bq[...]
    k = jnp.dot(xx, wk[...], preferred_element_type=jnp.float32) + bk[...]
    v = jnp.dot(xx, wv[...], preferred_element_type=jnp.float32) + bv[...]
    scores = (q * k) * (1.0 / 8.0)
    scores = scores - jnp.max(scores, axis=-1, keepdims=True)
    ex = jnp.exp(scores)
    attn = ex / jnp.sum(ex, axis=-1, keepdims=True)
    attn_out = attn * v
    mu = jnp.mean(attn_out, axis=-1, keepdims=True)
    ctr = attn_out - mu
    var = jnp.mean(ctr * ctr, axis=-1, keepdims=True)
    xn = ctr * lax.rsqrt(var + 1e-5) * gma[...] + bta[...]
    h = attn_out + jnp.maximum(
        jnp.dot(xn, wr[...], preferred_element_type=jnp.float32) + br[...], 0.0)
    gate = jnp.dot(h, ws[...], preferred_element_type=jnp.float32) + bs[...] \
        + attn_out
    out_ref[...] = xs - xx + jnp.dot(gate, wsc[...],
                                     preferred_element_type=jnp.float32)


def _dense(x_pad, xs, hp, w_scale):
    blk = 1280
    grid = NPAD // blk

    def row_spec(dim):
        return pl.BlockSpec((blk, dim), lambda i: (i, 0))

    def full_spec(a):
        return pl.BlockSpec(a.shape, lambda i: (0,) * a.ndim)

    weights = [hp['Wq'], hp['bq'].reshape(1, G), hp['Wk'], hp['bk'].reshape(1, G),
               hp['Wv'], hp['bv'].reshape(1, G), hp['res'][0]['W'],
               hp['res'][0]['b'].reshape(1, G), hp['res'][0]['gamma'].reshape(1, G),
               hp['res'][0]['beta'].reshape(1, G), hp['Ws'],
               hp['bs'].reshape(1, G), w_scale]
    return pl.pallas_call(
        _dense_body,
        out_shape=jax.ShapeDtypeStruct((NPAD, D), jnp.float32),
        grid=(grid,),
        in_specs=[row_spec(D), row_spec(D)] + [full_spec(w) for w in weights],
        out_specs=row_spec(D),
    )(x_pad, xs, *weights)


@functools.partial(
    pl.kernel,
    out_type=jax.ShapeDtypeStruct((N_EDGES, D), jnp.float32),
    mesh=_mesh,
    scratch_types=[
        pltpu.VMEM((EPT,), jnp.int32),
        pltpu.VMEM((GC, D), jnp.float32),
        pltpu.VMEM((GC, D), jnp.float32),
        pltpu.VMEM_SHARED((NPAD, D), jnp.float32),
        pltpu.SemaphoreType.DMA,
        pltpu.SemaphoreType.DMA,
        pltpu.SemaphoreType.DMA,
        pltpu.SemaphoreType.DMA,
        pltpu.SemaphoreType.DMA,
    ],
)
def _edge_gather(h_hbm, dst_hbm, out_hbm, dst_v, rows_a, rows_b, h_sh,
                 sem_l, sem_ga, sem_gb, sem_sa, sem_sb):
    wid = _wid()
    sid = lax.axis_index("s")
    base_e = wid * EPT
    rows_per_sub = NPAD // NS
    pltpu.async_copy(dst_hbm.at[pl.ds(base_e, EPT)], dst_v, sem_l)
    pltpu.sync_copy(h_hbm.at[pl.ds(sid * rows_per_sub, rows_per_sub)],
                    h_sh.at[pl.ds(sid * rows_per_sub, rows_per_sub)])
    pltpu.make_async_copy(dst_hbm.at[pl.ds(base_e, EPT)], dst_v, sem_l).wait()
    plsc.subcore_barrier()

    def g_start(c, buf, sem):
        pltpu.async_copy(h_sh.at[dst_v.at[pl.ds(c * GC, GC)]], buf, sem)

    def g_wait(buf, sem):
        pltpu.make_async_copy(h_sh.at[dst_v.at[pl.ds(0, GC)]], buf, sem).wait()

    def s_start(c, buf, sem):
        pltpu.async_copy(buf, out_hbm.at[pl.ds(base_e + c * GC, GC)], sem)

    def s_wait(buf, sem):
        pltpu.make_async_copy(buf, out_hbm.at[pl.ds(base_e, GC)], sem).wait()

    g_start(0, rows_a, sem_ga)

    def body(i, _):
        c = 2 * i

        @pl.when(i > 0)
        def _():
            s_wait(rows_b, sem_sb)

        g_start(c + 1, rows_b, sem_gb)
        g_wait(rows_a, sem_ga)
        s_start(c, rows_a, sem_sa)
        s_wait(rows_a, sem_sa)
        g_start(c + 2, rows_a, sem_ga)
        g_wait(rows_b, sem_gb)
        s_start(c + 1, rows_b, sem_sb)
        return 0

    n_pairs = (EPT // GC) // 2
    lax.fori_loop(0, n_pairs, body, 0)
    s_wait(rows_b, sem_sb)
    g_wait(rows_a, sem_ga)
    s_start(EPT // GC - 1, rows_a, sem_sa)
    s_wait(rows_a, sem_sa)


def kernel(x, edge_index, params):
    src = edge_index[0]
    dst = edge_index[1]
    hp = params['heads'][0]
    x_pad = jnp.concatenate(
        [x, jnp.zeros((NPAD - N_NODES, D), jnp.float32)], axis=0)

    tables = _win_tables(dst)
    xs = _merge_gather(tables, src, x)
    h_node = _dense(x_pad, xs, hp, params['W_scale'])
    return _edge_gather(h_node, dst)


# speedup vs baseline: 24.5020x; 1.0703x over previous
import functools

import jax
import jax.numpy as jnp
from jax import lax
from jax.experimental import pallas as pl
from jax.experimental.pallas import tpu as pltpu
from jax.experimental.pallas import tpu_sc as plsc

N_NODES = 10000
N_EDGES = 320000
D = 128
G = 64

NC, NS, L = 2, 16, 16
NW = NC * NS
NPAD = 10240
NPT = NPAD // NW
EPT = N_EDGES // NW
GC = 80

_mesh = plsc.VectorSubcoreMesh(core_axis_name="c", subcore_axis_name="s")


def _wid():
    return lax.axis_index("s") * NC + lax.axis_index("c")


@functools.partial(
    pl.kernel,
    out_type=jax.ShapeDtypeStruct((NW * NPAD,), jnp.int32),
    mesh=_mesh,
    compiler_params=pltpu.CompilerParams(needs_layout_passes=False),
    scratch_types=[
        pltpu.VMEM((EPT,), jnp.int32),
        pltpu.VMEM((NPAD,), jnp.int32),
        pltpu.VMEM((L,), jnp.int32),
    ],
)
def _win_tables(dst_hbm, tables_hbm, dst_v, win_v, tmp_v):
    wid = _wid()
    base_e = wid * EPT
    pltpu.sync_copy(dst_hbm.at[pl.ds(base_e, EPT)], dst_v)

    lane = lax.iota(jnp.int32, L)
    neg1 = jnp.full((L,), -1, jnp.int32)

    def init_body(i, _):
        win_v[pl.ds(i * L, L)] = neg1
        return 0

    lax.fori_loop(0, NPAD // L, init_body, 0)

    def body(c, _):
        d16 = dst_v[pl.ds(c * L, L)]
        e16 = base_e + c * L + lane
        plsc.store_scatter(win_v, [d16], e16)
        return 0

    lax.fori_loop(0, EPT // L, body, 0)
    pltpu.sync_copy(win_v, tables_hbm.at[pl.ds(wid * NPAD, NPAD)])


@functools.partial(
    pl.kernel,
    out_type=jax.ShapeDtypeStruct((NPAD, D), jnp.float32),
    mesh=_mesh,
    scratch_types=[
        pltpu.VMEM((NW * NPT,), jnp.int32),
        pltpu.VMEM((NPT,), jnp.int32),
        pltpu.VMEM((NPT, D), jnp.float32),
        pltpu.SemaphoreType.DMA,
        pltpu.SemaphoreType.DMA,
    ],
)
def _merge_gather(tables_hbm, src_hbm, x_hbm, xs_hbm, tabs_v, sidx_v,
                  xrows_v, sem_a, sem_b):
    wid = _wid()
    base_n = wid * NPT
    for t in range(NW):
        pltpu.async_copy(tables_hbm.at[pl.ds(t * NPAD + base_n, NPT)],
                         tabs_v.at[pl.ds(t * NPT, NPT)], sem_a)
    for t in range(NW):
        pltpu.make_async_copy(tables_hbm.at[pl.ds(base_n, NPT)],
                              tabs_v.at[pl.ds(0, NPT)], sem_a).wait()

    def merge_body(i, _):
        m = tabs_v[pl.ds(i * L, L)]
        for t in range(1, NW):
            m = jnp.maximum(m, tabs_v[pl.ds(t * NPT + i * L, L)])
        tabs_v[pl.ds(i * L, L)] = jnp.maximum(m, 0)
        return 0

    lax.fori_loop(0, NPT // L, merge_body, 0)

    for c in range(NPT // GC):
        pltpu.async_copy(src_hbm.at[tabs_v.at[pl.ds(c * GC, GC)]],
                         sidx_v.at[pl.ds(c * GC, GC)], sem_a)
    for c in range(NPT // GC):
        pltpu.make_async_copy(src_hbm.at[tabs_v.at[pl.ds(0, GC)]],
                              sidx_v.at[pl.ds(0, GC)], sem_a).wait()
    for c in range(NPT // GC):
        pltpu.async_copy(x_hbm.at[sidx_v.at[pl.ds(c * GC, GC)]],
                         xrows_v.at[pl.ds(c * GC, GC)], sem_b)
    for c in range(NPT // GC):
        pltpu.make_async_copy(x_hbm.at[sidx_v.at[pl.ds(0, GC)]],
                              xrows_v.at[pl.ds(0, GC)], sem_b).wait()
    pltpu.sync_copy(xrows_v, xs_hbm.at[pl.ds(base_n, NPT)])


def _dense_body(x_ref, xs_ref, wq, bq, wk, bk, wv, bv, wr, br, gma, bta, ws,
                bs, wsc, out_ref):
    xs = xs_ref[...]
    xx = x_ref[...]
    q = jnp.dot(xs, wq[...], preferred_element_type=jnp.float32) + bq[...]
    k = jnp.dot(xx, wk[...], preferred_element_type=jnp.float32) + bk[...]
    v = jnp.dot(xx, wv[...], preferred_element_type=jnp.float32) + bv[...]
    scores = (q * k) * (1.0 / 8.0)
    scores = scores - jnp.max(scores, axis=-1, keepdims=True)
    ex = jnp.exp(scores)
    attn = ex / jnp.sum(ex, axis=-1, keepdims=True)
    attn_out = attn * v
    mu = jnp.mean(attn_out, axis=-1, keepdims=True)
    ctr = attn_out - mu
    var = jnp.mean(ctr * ctr, axis=-1, keepdims=True)
    xn = ctr * lax.rsqrt(var + 1e-5) * gma[...] + bta[...]
    h = attn_out + jnp.maximum(
        jnp.dot(xn, wr[...], preferred_element_type=jnp.float32) + br[...], 0.0)
    gate = jnp.dot(h, ws[...], preferred_element_type=jnp.float32) + bs[...] \
        + attn_out
    out_ref[...] = xs - xx + jnp.dot(gate, wsc[...],
                                     preferred_element_type=jnp.float32)


def _dense(x_pad, xs, hp, w_scale):
    blk = 1280
    grid = NPAD // blk

    def row_spec(dim):
        return pl.BlockSpec((blk, dim), lambda i: (i, 0))

    def full_spec(a):
        return pl.BlockSpec(a.shape, lambda i: (0,) * a.ndim)

    weights = [hp['Wq'], hp['bq'].reshape(1, G), hp['Wk'], hp['bk'].reshape(1, G),
               hp['Wv'], hp['bv'].reshape(1, G), hp['res'][0]['W'],
               hp['res'][0]['b'].reshape(1, G), hp['res'][0]['gamma'].reshape(1, G),
               hp['res'][0]['beta'].reshape(1, G), hp['Ws'],
               hp['bs'].reshape(1, G), w_scale]
    return pl.pallas_call(
        _dense_body,
        out_shape=jax.ShapeDtypeStruct((NPAD, D), jnp.float32),
        grid=(grid,),
        in_specs=[row_spec(D), row_spec(D)] + [full_spec(w) for w in weights],
        out_specs=row_spec(D),
    )(x_pad, xs, *weights)


@functools.partial(
    pl.kernel,
    out_type=jax.ShapeDtypeStruct((N_EDGES, D), jnp.float32),
    mesh=_mesh,
    scratch_types=[
        pltpu.VMEM((EPT,), jnp.int32),
        pltpu.VMEM((GC, D), jnp.float32),
        pltpu.VMEM((GC, D), jnp.float32),
        pltpu.VMEM_SHARED((NPAD, D), jnp.float32),
        pltpu.SemaphoreType.DMA,
        pltpu.SemaphoreType.DMA,
        pltpu.SemaphoreType.DMA,
        pltpu.SemaphoreType.DMA,
        pltpu.SemaphoreType.DMA,
    ],
)
def _edge_gather(h_hbm, dst_hbm, out_hbm, dst_v, rows_a, rows_b, h_sh,
                 sem_l, sem_ga, sem_gb, sem_sa, sem_sb):
    wid = _wid()
    sid = lax.axis_index("s")
    base_e = wid * EPT
    rows_per_sub = NPAD // NS
    pltpu.async_copy(dst_hbm.at[pl.ds(base_e, EPT)], dst_v, sem_l)
    pltpu.sync_copy(h_hbm.at[pl.ds(sid * rows_per_sub, rows_per_sub)],
                    h_sh.at[pl.ds(sid * rows_per_sub, rows_per_sub)])
    pltpu.make_async_copy(dst_hbm.at[pl.ds(base_e, EPT)], dst_v, sem_l).wait()
    plsc.subcore_barrier()

    def g_start(c, buf, sem):
        pltpu.async_copy(h_sh.at[dst_v.at[pl.ds(c * GC, GC)]], buf, sem)

    def g_wait(buf, sem):
        pltpu.make_async_copy(h_sh.at[dst_v.at[pl.ds(0, GC)]], buf, sem).wait()

    def s_start(c, buf, sem):
        pltpu.async_copy(buf, out_hbm.at[pl.ds(base_e + c * GC, GC)], sem)

    def s_wait(buf, sem):
        pltpu.make_async_copy(buf, out_hbm.at[pl.ds(base_e, GC)], sem).wait()

    g_start(0, rows_a, sem_ga)

    def body(i, _):
        c = 2 * i

        @pl.when(i > 0)
        def _():
            s_wait(rows_b, sem_sb)

        g_start(c + 1, rows_b, sem_gb)
        g_wait(rows_a, sem_ga)
        s_start(c, rows_a, sem_sa)
        s_wait(rows_a, sem_sa)
        g_start(c + 2, rows_a, sem_ga)
        g_wait(rows_b, sem_gb)
        s_start(c + 1, rows_b, sem_sb)
        return 0

    n_pairs = (EPT // GC) // 2
    lax.fori_loop(0, n_pairs, body, 0)
    s_wait(rows_b, sem_sb)
    g_wait(rows_a, sem_ga)
    s_start(EPT // GC - 1, rows_a, sem_sa)
    s_wait(rows_a, sem_sa)


def kernel(x, edge_index, params):
    src = edge_index[0]
    dst = edge_index[1]
    hp = params['heads'][0]
    x_pad = jnp.concatenate(
        [x, jnp.zeros((NPAD - N_NODES, D), jnp.float32)], axis=0)

    tables = _win_tables(dst)
    xs = _merge_gather(tables, src, x)
    h_node = _dense(x_pad, xs, hp, params['W_scale'])
    return _edge_gather(h_node, dst)
